# baseline (speedup 1.0000x reference)
import functools

import jax
import jax.numpy as jnp
from jax import lax
from jax.experimental import pallas as pl
from jax.experimental.pallas import tpu as pltpu
from jax.experimental.pallas import tpu_sc as plsc

HP = 128
HPK = 64
CP = 64


def _gw_body(tt_ref, w_ref, b_ref, o_ref):
    o_ref[...] = (
        lax.dot_general(
            tt_ref[...], w_ref[...],
            dimension_numbers=(((0,), (0,)), ((), ())),
            preferred_element_type=jnp.float32,
        )
        + b_ref[...]
    )


def _table_times_w(tableT, Wp, bp, vblk=2048):
    E, V = tableT.shape
    return pl.pallas_call(
        _gw_body,
        grid=(pl.cdiv(V, vblk),),
        in_specs=[
            pl.BlockSpec((E, vblk), lambda i: (0, i)),
            pl.BlockSpec((E, HP), lambda i: (0, 0)),
            pl.BlockSpec((1, HP), lambda i: (0, 0)),
        ],
        out_specs=pl.BlockSpec((vblk, HP), lambda i: (i, 0)),
        out_shape=jax.ShapeDtypeStruct((V, HP), jnp.float32),
    )(tableT, Wp, bp)


def _make_gather(LB, nc, ns, k_grp=5):
    nw = nc * ns
    per_w = LB // nw
    grp = k_grp * 128
    n_groups = per_w // grp
    assert per_w % grp == 0

    mesh = plsc.VectorSubcoreMesh(core_axis_name="c", subcore_axis_name="s")

    @functools.partial(
        pl.kernel,
        mesh=mesh,
        out_type=[
            jax.ShapeDtypeStruct((LB, HP), jnp.float32),
            jax.ShapeDtypeStruct((nw, per_w // 128, 128), jnp.int32),
        ],
        scratch_types=[
            pltpu.VMEM((per_w // 128, 128), jnp.int32),
            pltpu.VMEM((grp, HP), jnp.float32),
            pltpu.SemaphoreType.DMA,
        ],
    )
    def gather_k(g_hbm, idx_hbm, out_hbm, mask_hbm, idx_v, rows_v, sem):
        wid = lax.axis_index("s") * nc + lax.axis_index("c")
        base = wid * per_w
        pltpu.sync_copy(idx_hbm.at[wid], idx_v)
        pltpu.sync_copy(idx_v, mask_hbm.at[wid])
        for g in range(n_groups):
            off = base + g * grp
            descs = [
                pltpu.async_copy(
                    g_hbm.at[idx_v.at[g * k_grp + j]],
                    rows_v.at[pl.ds(j * 128, 128)],
                    sem,
                )
                for j in range(k_grp)
            ]
            for d in descs:
                d.wait()
            pltpu.sync_copy(rows_v, out_hbm.at[pl.ds(off, grp)])

    return gather_k


def _scan_body(n_steps, t_blk, emit_probs, xb_ref, idx_ref, u_ref, wfc_ref,
               bfc_ref, hin_ref, o_ref, h_ref):
    step = pl.program_id(0)

    @pl.when(step == 0)
    def _init():
        h_ref[...] = hin_ref[...]

    HB = xb_ref.shape[1] // 2
    ha = h_ref[: HB, :]
    hb = h_ref[HB:, :]
    u = u_ref[...]
    for t in range(t_blk):
        xa = xb_ref[t, :HB, :]
        xb = xb_ref[t, HB:, :]
        ma = idx_ref[t, :HB, :] != 0
        mb = idx_ref[t, HB:, :] != 0
        hna = jnp.tanh(xa + jnp.dot(ha, u, preferred_element_type=jnp.float32))
        hnb = jnp.tanh(xb + jnp.dot(hb, u, preferred_element_type=jnp.float32))
        ha = jnp.where(ma, hna, ha)
        hb = jnp.where(mb, hnb, hb)
    h_ref[: HB, :] = ha
    h_ref[HB:, :] = hb

    if emit_probs:
        @pl.when(step == n_steps - 1)
        def _tail():
            h = jnp.concatenate([ha, hb], axis=0)
            logits = (
                jnp.dot(h, wfc_ref[...], preferred_element_type=jnp.float32)
                + bfc_ref[...]
            )
            mx = jnp.max(logits, axis=-1, keepdims=True)
            e = jnp.exp(logits - mx)
            o_ref[...] = e / jnp.sum(e, axis=-1, keepdims=True)
    else:
        o_ref[: HB, :] = ha
        o_ref[HB:, :] = hb


def _rnn_scan(xb3, idx3, Up, Wfcp, bfcp, h0, emit_probs, t_blk=10):
    Lc, B, _ = xb3.shape
    assert Lc % t_blk == 0
    n_steps = Lc // t_blk
    oc = CP if emit_probs else HP
    return pl.pallas_call(
        functools.partial(_scan_body, n_steps, t_blk, emit_probs),
        grid=(n_steps,),
        in_specs=[
            pl.BlockSpec((t_blk, B, HP), lambda i: (i, 0, 0)),
            pl.BlockSpec((t_blk, B, 1), lambda i: (i, 0, 0)),
            pl.BlockSpec((HP, HP), lambda i: (0, 0)),
            pl.BlockSpec((HP, CP), lambda i: (0, 0)),
            pl.BlockSpec((1, CP), lambda i: (0, 0)),
            pl.BlockSpec((B, HP), lambda i: (0, 0)),
        ],
        out_specs=pl.BlockSpec((B, oc), lambda i: (0, 0)),
        out_shape=jax.ShapeDtypeStruct((B, oc), jnp.float32),
        scratch_shapes=[pltpu.VMEM((B, HP), jnp.float32)],
    )(xb3, idx3, Up, Wfcp, bfcp, h0)


def kernel(indices, table, W, U, b, Wfc, bfc):
    B, L = indices.shape
    V, E = table.shape
    H = W.shape[1]
    C = Wfc.shape[1]

    Wp = jnp.pad(W, ((0, 0), (0, HP - H)))
    bp = jnp.pad(b, (0, HP - H)).reshape(1, HP)
    Up = jnp.pad(U, ((0, HP - H), (0, HP - H)))
    Wfcp = jnp.pad(Wfc, ((0, HP - H), (0, CP - C)))
    bfcp = jnp.pad(bfc, (0, CP - C), constant_values=-1e30).reshape(1, CP)

    G = _table_times_w(jnp.swapaxes(table, 0, 1), Wp, bp)

    info = plsc.get_sparse_core_info()
    nw = info.num_cores * info.num_subcores
    idxT = jnp.swapaxes(indices, 0, 1)

    Lh = L // 2
    LBh = Lh * B
    gather_k = _make_gather(LBh, info.num_cores, info.num_subcores)
    idx2a = idxT[:Lh].reshape(nw, LBh // nw // 128, 128).astype(jnp.int32)
    idx2b = idxT[Lh:].reshape(nw, LBh // nw // 128, 128).astype(jnp.int32)
    xb_a, midx_a = gather_k(G, idx2a)
    xb_b, midx_b = gather_k(G, idx2b)
    xb_a = xb_a.reshape(Lh, B, HP)
    xb_b = xb_b.reshape(Lh, B, HP)
    mask_a = midx_a.reshape(Lh, B, 1)
    mask_b = midx_b.reshape(Lh, B, 1)
    h0 = jnp.zeros((B, HP), jnp.float32)
    hmid = _rnn_scan(xb_a, mask_a, Up, Wfcp, bfcp, h0, False)
    probs = _rnn_scan(xb_b, mask_b, Up, Wfcp, bfcp, hmid, True)
    return probs[:, :C]


# speedup vs baseline: 1.0022x; 1.0022x over previous
import functools

import jax
import jax.numpy as jnp
from jax import lax
from jax.experimental import pallas as pl
from jax.experimental.pallas import tpu as pltpu
from jax.experimental.pallas import tpu_sc as plsc

HP = 128
HPK = 64
CP = 64


def _gw_body(tt_ref, w_ref, b_ref, o_ref):
    o_ref[...] = (
        lax.dot_general(
            tt_ref[...], w_ref[...],
            dimension_numbers=(((0,), (0,)), ((), ())),
            preferred_element_type=jnp.float32,
        )
        + b_ref[...]
    )


def _table_times_w(tableT, Wp, bp, vblk=2048):
    E, V = tableT.shape
    return pl.pallas_call(
        _gw_body,
        grid=(pl.cdiv(V, vblk),),
        in_specs=[
            pl.BlockSpec((E, vblk), lambda i: (0, i)),
            pl.BlockSpec((E, HP), lambda i: (0, 0)),
            pl.BlockSpec((1, HP), lambda i: (0, 0)),
        ],
        out_specs=pl.BlockSpec((vblk, HP), lambda i: (i, 0)),
        out_shape=jax.ShapeDtypeStruct((V, HP), jnp.float32),
    )(tableT, Wp, bp)


def _make_gather(LB, nc, ns, k_grp=5):
    nw = nc * ns
    per_w = LB // nw
    grp = k_grp * 128
    n_groups = per_w // grp
    assert per_w % grp == 0

    mesh = plsc.VectorSubcoreMesh(core_axis_name="c", subcore_axis_name="s")

    @functools.partial(
        pl.kernel,
        mesh=mesh,
        out_type=[
            jax.ShapeDtypeStruct((LB, HP), jnp.float32),
            jax.ShapeDtypeStruct((LB,), jnp.int32),
        ],
        scratch_types=[
            pltpu.VMEM((per_w // 128, 128), jnp.int32),
            pltpu.VMEM((grp, HP), jnp.float32),
            pltpu.SemaphoreType.DMA,
        ],
    )
    def gather_k(g_hbm, idx_hbm, out_hbm, mask_hbm, idx_v, rows_v, sem):
        wid = lax.axis_index("s") * nc + lax.axis_index("c")
        base = wid * per_w
        pltpu.sync_copy(idx_hbm.at[wid], idx_v)
        for r in range(per_w // 128):
            pltpu.sync_copy(idx_v.at[r], mask_hbm.at[pl.ds(base + r * 128, 128)])
        for g in range(n_groups):
            off = base + g * grp
            descs = [
                pltpu.async_copy(
                    g_hbm.at[idx_v.at[g * k_grp + j]],
                    rows_v.at[pl.ds(j * 128, 128)],
                    sem,
                )
                for j in range(k_grp)
            ]
            for d in descs:
                d.wait()
            pltpu.sync_copy(rows_v, out_hbm.at[pl.ds(off, grp)])

    return gather_k


def _scan_body(n_steps, t_blk, emit_probs, xb_ref, idx_ref, u_ref, wfc_ref,
               bfc_ref, hin_ref, o_ref, h_ref):
    step = pl.program_id(0)

    @pl.when(step == 0)
    def _init():
        h_ref[...] = hin_ref[...]

    HB = xb_ref.shape[1] // 2
    ha = h_ref[: HB, :]
    hb = h_ref[HB:, :]
    u = u_ref[...]
    for t in range(t_blk):
        xa = xb_ref[t, :HB, :]
        xb = xb_ref[t, HB:, :]
        ma = idx_ref[t, :HB, :] != 0
        mb = idx_ref[t, HB:, :] != 0
        hna = jnp.tanh(xa + jnp.dot(ha, u, preferred_element_type=jnp.float32))
        hnb = jnp.tanh(xb + jnp.dot(hb, u, preferred_element_type=jnp.float32))
        ha = jnp.where(ma, hna, ha)
        hb = jnp.where(mb, hnb, hb)
    h_ref[: HB, :] = ha
    h_ref[HB:, :] = hb

    if emit_probs:
        @pl.when(step == n_steps - 1)
        def _tail():
            h = jnp.concatenate([ha, hb], axis=0)
            logits = (
                jnp.dot(h, wfc_ref[...], preferred_element_type=jnp.float32)
                + bfc_ref[...]
            )
            mx = jnp.max(logits, axis=-1, keepdims=True)
            e = jnp.exp(logits - mx)
            o_ref[...] = e / jnp.sum(e, axis=-1, keepdims=True)
    else:
        o_ref[: HB, :] = ha
        o_ref[HB:, :] = hb


def _rnn_scan(xb3, idx3, Up, Wfcp, bfcp, h0, emit_probs, t_blk=10):
    Lc, B, _ = xb3.shape
    assert Lc % t_blk == 0
    n_steps = Lc // t_blk
    oc = CP if emit_probs else HP
    return pl.pallas_call(
        functools.partial(_scan_body, n_steps, t_blk, emit_probs),
        grid=(n_steps,),
        in_specs=[
            pl.BlockSpec((t_blk, B, HP), lambda i: (i, 0, 0)),
            pl.BlockSpec((t_blk, B, 1), lambda i: (i, 0, 0)),
            pl.BlockSpec((HP, HP), lambda i: (0, 0)),
            pl.BlockSpec((HP, CP), lambda i: (0, 0)),
            pl.BlockSpec((1, CP), lambda i: (0, 0)),
            pl.BlockSpec((B, HP), lambda i: (0, 0)),
        ],
        out_specs=pl.BlockSpec((B, oc), lambda i: (0, 0)),
        out_shape=jax.ShapeDtypeStruct((B, oc), jnp.float32),
        scratch_shapes=[pltpu.VMEM((B, HP), jnp.float32)],
    )(xb3, idx3, Up, Wfcp, bfcp, h0)


def kernel(indices, table, W, U, b, Wfc, bfc):
    B, L = indices.shape
    V, E = table.shape
    H = W.shape[1]
    C = Wfc.shape[1]

    Wp = jnp.pad(W, ((0, 0), (0, HP - H)))
    bp = jnp.pad(b, (0, HP - H)).reshape(1, HP)
    Up = jnp.pad(U, ((0, HP - H), (0, HP - H)))
    Wfcp = jnp.pad(Wfc, ((0, HP - H), (0, CP - C)))
    bfcp = jnp.pad(bfc, (0, CP - C), constant_values=-1e30).reshape(1, CP)

    G = _table_times_w(jnp.swapaxes(table, 0, 1), Wp, bp)

    info = plsc.get_sparse_core_info()
    nw = info.num_cores * info.num_subcores
    idxT = jnp.swapaxes(indices, 0, 1)

    Lh = L // 2
    LBh = Lh * B
    gather_k = _make_gather(LBh, info.num_cores, info.num_subcores)
    idx2a = idxT[:Lh].reshape(nw, LBh // nw // 128, 128).astype(jnp.int32)
    idx2b = idxT[Lh:].reshape(nw, LBh // nw // 128, 128).astype(jnp.int32)
    xb_a, midx_a = gather_k(G, idx2a)
    xb_b, midx_b = gather_k(G, idx2b)
    xb_a = xb_a.reshape(Lh, B, HP)
    xb_b = xb_b.reshape(Lh, B, HP)
    mask_a = midx_a.reshape(Lh, B, 1)
    mask_b = midx_b.reshape(Lh, B, 1)
    h0 = jnp.zeros((B, HP), jnp.float32)
    hmid = _rnn_scan(xb_a, mask_a, Up, Wfcp, bfcp, h0, False)
    probs = _rnn_scan(xb_b, mask_b, Up, Wfcp, bfcp, hmid, True)
    return probs[:, :C]


# speedup vs baseline: 1.4574x; 1.4543x over previous
import functools

import jax
import jax.numpy as jnp
from jax import lax
from jax.experimental import pallas as pl
from jax.experimental.pallas import tpu as pltpu
from jax.experimental.pallas import tpu_sc as plsc

HP = 128
HPK = 64
CP = 64


def _gw_body(vblk, flag_lane, tt_ref, w_ref, b_ref, o_ref):
    g = (
        lax.dot_general(
            tt_ref[...], w_ref[...],
            dimension_numbers=(((0,), (0,)), ((), ())),
            preferred_element_type=jnp.float32,
        )
        + b_ref[...]
    )
    rids = pl.program_id(0) * vblk + lax.broadcasted_iota(
        jnp.int32, g.shape, 0
    )
    lids = lax.broadcasted_iota(jnp.int32, g.shape, 1)
    flag = (rids != 0).astype(jnp.float32)
    o_ref[...] = jnp.where(lids == flag_lane, flag, g)


def _table_times_w(tableT, Wp, bp, flag_lane, vblk=2048):
    E, V = tableT.shape
    return pl.pallas_call(
        functools.partial(_gw_body, vblk, flag_lane),
        grid=(pl.cdiv(V, vblk),),
        in_specs=[
            pl.BlockSpec((E, vblk), lambda i: (0, i)),
            pl.BlockSpec((E, HP), lambda i: (0, 0)),
            pl.BlockSpec((1, HP), lambda i: (0, 0)),
        ],
        out_specs=pl.BlockSpec((vblk, HP), lambda i: (i, 0)),
        out_shape=jax.ShapeDtypeStruct((V, HP), jnp.float32),
    )(tableT, Wp, bp)


def _make_gather(LB, nc, ns, k_grp=5):
    nw = nc * ns
    per_w = LB // nw
    grp = k_grp * 128
    n_groups = per_w // grp
    assert per_w % grp == 0

    mesh = plsc.VectorSubcoreMesh(core_axis_name="c", subcore_axis_name="s")

    @functools.partial(
        pl.kernel,
        mesh=mesh,
        out_type=jax.ShapeDtypeStruct((LB, HP), jnp.float32),
        scratch_types=[
            pltpu.VMEM((per_w // 128, 128), jnp.int32),
            pltpu.VMEM((grp, HP), jnp.float32),
            pltpu.SemaphoreType.DMA,
        ],
    )
    def gather_k(g_hbm, idx_hbm, out_hbm, idx_v, rows_v, sem):
        wid = lax.axis_index("s") * nc + lax.axis_index("c")
        base = wid * per_w
        pltpu.sync_copy(idx_hbm.at[wid], idx_v)
        for g in range(n_groups):
            off = base + g * grp
            descs = [
                pltpu.async_copy(
                    g_hbm.at[idx_v.at[g * k_grp + j]],
                    rows_v.at[pl.ds(j * 128, 128)],
                    sem,
                )
                for j in range(k_grp)
            ]
            for d in descs:
                d.wait()
            pltpu.sync_copy(rows_v, out_hbm.at[pl.ds(off, grp)])

    return gather_k


def _scan_body(n_steps, t_blk, emit_probs, flag_lane, xb_ref, u_ref, wfc_ref,
               bfc_ref, hin_ref, o_ref, h_ref):
    step = pl.program_id(0)

    @pl.when(step == 0)
    def _init():
        h_ref[...] = hin_ref[...]

    HB = xb_ref.shape[1] // 2
    ha = h_ref[: HB, :]
    hb = h_ref[HB:, :]
    u = u_ref[...]
    for t in range(t_blk):
        xa = xb_ref[t, :HB, :]
        xb = xb_ref[t, HB:, :]
        ma = xa[:, flag_lane:flag_lane + 1] != 0.0
        mb = xb[:, flag_lane:flag_lane + 1] != 0.0
        hna = jnp.tanh(xa + jnp.dot(ha, u, preferred_element_type=jnp.float32))
        hnb = jnp.tanh(xb + jnp.dot(hb, u, preferred_element_type=jnp.float32))
        ha = jnp.where(ma, hna, ha)
        hb = jnp.where(mb, hnb, hb)
    h_ref[: HB, :] = ha
    h_ref[HB:, :] = hb

    if emit_probs:
        @pl.when(step == n_steps - 1)
        def _tail():
            h = jnp.concatenate([ha, hb], axis=0)
            logits = (
                jnp.dot(h, wfc_ref[...], preferred_element_type=jnp.float32)
                + bfc_ref[...]
            )
            mx = jnp.max(logits, axis=-1, keepdims=True)
            e = jnp.exp(logits - mx)
            o_ref[...] = e / jnp.sum(e, axis=-1, keepdims=True)
    else:
        o_ref[: HB, :] = ha
        o_ref[HB:, :] = hb


def _rnn_scan(xb3, Up, Wfcp, bfcp, h0, emit_probs, flag_lane, t_blk=10):
    Lc, B, _ = xb3.shape
    assert Lc % t_blk == 0
    n_steps = Lc // t_blk
    oc = CP if emit_probs else HP
    return pl.pallas_call(
        functools.partial(_scan_body, n_steps, t_blk, emit_probs, flag_lane),
        grid=(n_steps,),
        in_specs=[
            pl.BlockSpec((t_blk, B, HP), lambda i: (i, 0, 0)),
            pl.BlockSpec((HP, HP), lambda i: (0, 0)),
            pl.BlockSpec((HP, CP), lambda i: (0, 0)),
            pl.BlockSpec((1, CP), lambda i: (0, 0)),
            pl.BlockSpec((B, HP), lambda i: (0, 0)),
        ],
        out_specs=pl.BlockSpec((B, oc), lambda i: (0, 0)),
        out_shape=jax.ShapeDtypeStruct((B, oc), jnp.float32),
        scratch_shapes=[pltpu.VMEM((B, HP), jnp.float32)],
    )(xb3, Up, Wfcp, bfcp, h0)


def kernel(indices, table, W, U, b, Wfc, bfc):
    B, L = indices.shape
    V, E = table.shape
    H = W.shape[1]
    C = Wfc.shape[1]

    Wp = jnp.pad(W, ((0, 0), (0, HP - H)))
    bp = jnp.pad(b, (0, HP - H)).reshape(1, HP)
    Up = jnp.pad(U, ((0, HP - H), (0, HP - H)))
    Wfcp = jnp.pad(Wfc, ((0, HP - H), (0, CP - C)))
    bfcp = jnp.pad(bfc, (0, CP - C), constant_values=-1e30).reshape(1, CP)

    G = _table_times_w(jnp.swapaxes(table, 0, 1), Wp, bp, H)

    info = plsc.get_sparse_core_info()
    nw = info.num_cores * info.num_subcores
    idxT = jnp.swapaxes(indices, 0, 1)

    Lh = L // 2
    LBh = Lh * B
    gather_k = _make_gather(LBh, info.num_cores, info.num_subcores)
    idx2a = idxT[:Lh].reshape(nw, LBh // nw // 128, 128).astype(jnp.int32)
    idx2b = idxT[Lh:].reshape(nw, LBh // nw // 128, 128).astype(jnp.int32)
    xb_a = gather_k(G, idx2a).reshape(Lh, B, HP)
    xb_b = gather_k(G, idx2b).reshape(Lh, B, HP)

    h0 = jnp.zeros((B, HP), jnp.float32)
    hmid = _rnn_scan(xb_a, Up, Wfcp, bfcp, h0, False, H)
    probs = _rnn_scan(xb_b, Up, Wfcp, bfcp, hmid, True, H)
    return probs[:, :C]
